# baseline (speedup 1.0000x reference)
import functools

import jax
import jax.numpy as jnp
from jax import lax
from jax.experimental import pallas as pl
from jax.experimental.pallas import tpu as pltpu
from jax.experimental.pallas import tpu_sc as plsc

NN = 10000
EE = 330000
EP = 331776
EB = 64
NC = 2
NS = 16
NP = 10240
RPT = NP // NS
F32 = jnp.float32
I32 = jnp.int32


def _mm_body(x_ref, w_ref, o_ref):
    o_ref[...] = jnp.dot(x_ref[...], w_ref[...], preferred_element_type=F32)


def _mm(x, w, bm=512):
    m, k = x.shape
    _, n = w.shape
    return pl.pallas_call(
        _mm_body,
        grid=(m // bm,),
        in_specs=[pl.BlockSpec((bm, k), lambda i: (i, 0)),
                  pl.BlockSpec((k, n), lambda i: (0, 0))],
        out_specs=pl.BlockSpec((bm, n), lambda i: (i, 0)),
        out_shape=jax.ShapeDtypeStruct((m, n), F32),
    )(x, w)


def _mma_body(x_ref, w_ref, a_ref, g_ref):
    r = jnp.dot(x_ref[...], w_ref[...], preferred_element_type=F32)
    a_ref[...] = r
    rmax = jnp.max(r, axis=0, keepdims=True)

    @pl.when(pl.program_id(0) == 0)
    def _():
        g_ref[...] = rmax

    @pl.when(pl.program_id(0) != 0)
    def _():
        g_ref[...] = jnp.maximum(g_ref[...], rmax)


def _mma(x, w, bm=512):
    m, k = x.shape
    return pl.pallas_call(
        _mma_body,
        grid=(m // bm,),
        in_specs=[pl.BlockSpec((bm, k), lambda i: (i, 0)),
                  pl.BlockSpec((k, 128), lambda i: (0, 0))],
        out_specs=[pl.BlockSpec((bm, 128), lambda i: (i, 0)),
                   pl.BlockSpec((1, 128), lambda i: (0, 0))],
        out_shape=[jax.ShapeDtypeStruct((m, 128), F32),
                   jax.ShapeDtypeStruct((1, 128), F32)],
    )(x, w)


def _phaseB_body(src_hbm, dst_hbm, asd_hbm, gmax_hbm,
                 ex_hbm,
                 srcb, dstb, srows, drows, exb, gb):
    cid = lax.axis_index("c")
    sid = lax.axis_index("s")
    wid = sid * NC + cid

    pltpu.sync_copy(gmax_hbm, gb)

    nb = EP // (NC * NS) // 32
    base0 = wid * (EP // (NC * NS))

    def batch(b, _):
        base = base0 + b * 32
        pltpu.sync_copy(src_hbm.at[pl.ds(base, 32)], srcb)
        pltpu.sync_copy(dst_hbm.at[pl.ds(base, 32)], dstb)
        pltpu.sync_copy(asd_hbm.at[srcb], srows)
        pltpu.sync_copy(asd_hbm.at[dstb], drows)
        gv = gb[0, pl.ds(0, 16)]

        def edge(e, _):
            asv = srows[e, pl.ds(0, 16)]
            adv = drows[e, pl.ds(16, 16)]
            z = asv + adv
            z = jnp.where(z >= 0.0, z, 0.2 * z)
            mm = gv + adv
            mm = jnp.where(mm >= 0.0, mm, 0.2 * mm)
            exb[e, :] = jnp.exp(z - mm)
            return _
        lax.fori_loop(0, 32, edge, None)

        pltpu.sync_copy(exb, ex_hbm.at[pl.ds(base, 32)])
        return _
    lax.fori_loop(0, nb, batch, None)


_phaseB = pl.kernel(
    _phaseB_body,
    out_type=jax.ShapeDtypeStruct((EP, 16), F32),
    mesh=plsc.VectorSubcoreMesh(core_axis_name="c", subcore_axis_name="s"),
    scratch_types=[
        pltpu.VMEM((32,), I32), pltpu.VMEM((32,), I32),
        pltpu.VMEM((32, 128), F32), pltpu.VMEM((32, 128), F32),
        pltpu.VMEM((32, 16), F32),
        pltpu.VMEM((1, 128), F32),
    ])


def _phaseC_body(src_hbm, dst_hbm, ex_hbm, xw_hbm, meta_hbm,
                 acc_hbm, den_hbm,
                 acc_sp, idxb, exrb, hrow0, hrow1, zb,
                 sg0, sg1, ss0, ss1):
    cid = lax.axis_index("c")
    sid = lax.axis_index("s")
    srcs = (idxb.at[0], idxb.at[1])
    dsts = (idxb.at[2], idxb.at[3])
    exrs = (exrb.at[0], exrb.at[1])
    hrows = (hrow0, hrow1)
    sgs = (sg0, sg1)
    sss = (ss0, ss1)

    def zz(i, _):
        for j in range(8):
            zb[i, pl.ds(j * 16, 16)] = jnp.zeros((16,), F32)
        return _
    lax.fori_loop(0, 4, zz, None)

    pltpu.sync_copy(meta_hbm, zb.at[pl.ds(4, 1)])
    metav = zb[4, pl.ds(0, 16)].astype(I32)

    nb = EP // NS // EB
    base0 = sid * (EP // NS)

    def zero_acc():
        for z5 in range(160):
            pltpu.sync_copy(zb.at[pl.ds(0, 4)],
                            acc_sp.at[pl.ds(sid * RPT + z5 * 4, 4)])

    def _bcast16(v, idx):
        if isinstance(idx, jax.Array) and idx.ndim == 1:
            ind = idx[:, None]
        else:
            ind = jnp.full((16, 1), idx, I32)
        return lax.gather(
            v, ind,
            lax.GatherDimensionNumbers(
                offset_dims=(), collapsed_slice_dims=(0,),
                start_index_map=(0,)),
            (1,), mode=lax.GatherScatterMode.PROMISE_IN_BOUNDS)

    for k in range(4):
        cs = cid * 4 + k
        lanevec = _bcast16(metav, cs)

        zero_acc()
        plsc.subcore_barrier()

        def loads(b, s):
            base = base0 + b * EB
            pltpu.sync_copy(src_hbm.at[pl.ds(base, EB)], srcs[s])
            pltpu.sync_copy(dst_hbm.at[pl.ds(base, EB)], dsts[s])
            pltpu.sync_copy(ex_hbm.at[pl.ds(base, EB)], exrs[s])
            for c in range(EB // 16):
                sl = pl.ds(c * 16, 16)
                srcs[s][sl] = srcs[s][sl] * 16 + cs

        def g_issue(s):
            pltpu.async_copy(xw_hbm.at[srcs[s]], hrows[s], sgs[s])

        def g_wait(s):
            pltpu.make_async_copy(xw_hbm.at[srcs[s]], hrows[s], sgs[s]).wait()

        def s_issue(s):
            pltpu.async_copy(hrows[s], acc_sp.at[dsts[s]], sss[s], add=True)

        def s_wait(s):
            pltpu.make_async_copy(
                hrows[s], acc_sp.at[dsts[s]], sss[s]).wait()

        def scale(s):
            exr = exrs[s]
            hr = hrows[s]

            def edge4(q, _):
                for u in range(4):
                    e = q * 4 + u
                    alpha = _bcast16(exr[e, :], lanevec)
                    for j in range(8):
                        sl = pl.ds(j * 16, 16)
                        hr[e, sl] = hr[e, sl] * alpha
                return _
            lax.fori_loop(0, EB // 4, edge4, None)

        loads(0, 0)
        g_issue(0)

        def pair(p, _):
            b = 2 * p

            @pl.when(p >= 1)
            def _():
                s_wait(1)
            loads(b + 1, 1)
            g_issue(1)
            g_wait(0)
            scale(0)
            s_issue(0)

            g_wait(1)
            scale(1)
            s_issue(1)
            s_wait(0)
            loads(b + 2, 0)
            g_issue(0)
            return _
        lax.fori_loop(0, (nb - 2) // 2, pair, None)

        s_wait(1)
        loads(nb - 1, 1)
        g_issue(1)
        g_wait(0)
        scale(0)
        s_issue(0)
        g_wait(1)
        scale(1)
        s_issue(1)
        s_wait(0)
        s_wait(1)

        plsc.subcore_barrier()
        pltpu.sync_copy(acc_sp.at[pl.ds(sid * RPT, RPT)],
                        acc_hbm.at[cs, pl.ds(sid * RPT, RPT)])
        plsc.subcore_barrier()

    def zh(i, _):
        for j in range(8):
            hrow0[i, pl.ds(j * 16, 16)] = jnp.zeros((16,), F32)
            hrow1[i, pl.ds(j * 16, 16)] = jnp.zeros((16,), F32)
        return _
    lax.fori_loop(0, EB, zh, None)
    zero_acc()
    plsc.subcore_barrier()

    nb2 = EP // NS // EB
    base1 = sid * (EP // NS)

    def dloads(b, s):
        base = base1 + b * EB
        pltpu.sync_copy(dst_hbm.at[pl.ds(base, EB)], dsts[s])
        pltpu.sync_copy(ex_hbm.at[pl.ds(base, EB)], exrs[s])

    def dfill(s):
        exr = exrs[s]
        hr = hrows[s]

        def de(e, _):
            hr[e, pl.ds(0, 16)] = exr[e, :]
            return _
        lax.fori_loop(0, EB, de, None)

    def s_issue2(s):
        pltpu.async_copy(hrows[s], acc_sp.at[dsts[s]], sss[s], add=True)

    def s_wait2(s):
        pltpu.make_async_copy(
            hrows[s], acc_sp.at[dsts[s]], sss[s]).wait()

    dloads(0, 0)
    dfill(0)
    s_issue2(0)

    def dpair(p, _):
        b = 2 * p

        @pl.when(p >= 1)
        def _():
            s_wait2(1)
        dloads(b + 1, 1)
        dfill(1)
        s_issue2(1)
        s_wait2(0)
        dloads(b + 2, 0)
        dfill(0)
        s_issue2(0)
        return _
    lax.fori_loop(0, (nb2 - 2) // 2, dpair, None)

    s_wait2(1)
    dloads(nb2 - 1, 1)
    dfill(1)
    s_issue2(1)
    s_wait2(0)
    s_wait2(1)

    plsc.subcore_barrier()

    @pl.when(cid == 0)
    def _():
        pltpu.sync_copy(acc_sp.at[pl.ds(sid * RPT, RPT)],
                        den_hbm.at[pl.ds(sid * RPT, RPT)])


_phaseC = pl.kernel(
    _phaseC_body,
    out_type=[jax.ShapeDtypeStruct((8, NP, 128), F32),
              jax.ShapeDtypeStruct((NP, 128), F32)],
    mesh=plsc.VectorSubcoreMesh(core_axis_name="c", subcore_axis_name="s"),
    scratch_types=[
        pltpu.VMEM_SHARED((NP, 128), F32),
        pltpu.VMEM((4, EB), I32),
        pltpu.VMEM((2, EB, 16), F32),
        pltpu.VMEM((EB, 128), F32), pltpu.VMEM((EB, 128), F32),
        pltpu.VMEM((5, 128), F32),
        pltpu.SemaphoreType.DMA, pltpu.SemaphoreType.DMA,
        pltpu.SemaphoreType.DMA, pltpu.SemaphoreType.DMA,
    ])


def _fix12_body(acc_ref, den_ref, lin_ref, b_ref, o_ref):
    den = den_ref[...] + 1e-16
    for s in range(8):
        h = s // 2
        d = den[:, h:h + 1]
        t = acc_ref[s] / d + lin_ref[:, s * 128:(s + 1) * 128] \
            + b_ref[:, s * 128:(s + 1) * 128]
        o_ref[:, s * 128:(s + 1) * 128] = jnp.where(t > 0.0, t, jnp.exp(t) - 1.0)


def _fix12(acc, den, xw, bsum, bm=512):
    return pl.pallas_call(
        _fix12_body,
        grid=(NP // bm,),
        in_specs=[pl.BlockSpec((8, bm, 128), lambda i: (0, i, 0)),
                  pl.BlockSpec((bm, 128), lambda i: (i, 0)),
                  pl.BlockSpec((bm, 1024), lambda i: (i, 1)),
                  pl.BlockSpec((1, 1024), lambda i: (0, 0))],
        out_specs=pl.BlockSpec((bm, 1024), lambda i: (i, 0)),
        out_shape=jax.ShapeDtypeStruct((NP, 1024), F32),
    )(acc, den, xw, bsum)


def _fix3_body(acc_ref, den_ref, lin_ref, b_ref, o_ref):
    den = den_ref[...] + 1e-16
    t = jnp.zeros(acc_ref.shape[1:], F32)
    for s in range(6):
        t = t + acc_ref[s] / den[:, s:s + 1]
    t = t * (1.0 / 6.0) + lin_ref[...] + b_ref[...]
    o_ref[...] = t[:, :121]


def _fix3(acc, den, xw, bsum, bm=400):
    return pl.pallas_call(
        _fix3_body,
        grid=(NN // bm,),
        in_specs=[pl.BlockSpec((6, bm, 128), lambda i: (0, i, 0)),
                  pl.BlockSpec((bm, 128), lambda i: (i, 0)),
                  pl.BlockSpec((bm, 128), lambda i: (i, 6)),
                  pl.BlockSpec((1, 128), lambda i: (0, 0))],
        out_specs=pl.BlockSpec((bm, 121), lambda i: (i, 0)),
        out_shape=jax.ShapeDtypeStruct((NN, 121), F32),
    )(acc, den, xw, bsum)


def _wa_table(W, a_s, a_d, heads, ch, reps):
    ws = (W.reshape(-1, heads, ch) * a_s[None]).sum(-1)
    wd = (W.reshape(-1, heads, ch) * a_d[None]).sum(-1)
    ws16 = jnp.tile(ws, (1, reps))[:, :16]
    wd16 = jnp.tile(wd, (1, reps))[:, :16]
    pad = jnp.zeros((W.shape[0], 96), F32)
    return jnp.concatenate([ws16, wd16, pad], axis=1)


def _layer(x, whl, wa, bsum, src, dst, meta, fix):
    xw = _mm(x, whl)
    asd, gmax = _mma(x, wa)
    ex = _phaseB(src, dst, asd, gmax)
    xw_v = xw.reshape(16 * NP, 128)
    acc, den = _phaseC(src, dst, ex, xw_v, meta)
    return fix(acc, den, xw, bsum)


def kernel(x, edge_index, W1, a1s, a1d, b1, Wl1, bl1, W2, a2s, a2d, b2,
           Wl2, bl2, W3, a3s, a3d, b3, Wl3, bl3):
    loop = jnp.arange(NN, dtype=I32)
    npad = EP - EE
    pad_s = (jnp.arange(npad, dtype=I32) * 97) % NN
    pad_d = NN + (jnp.arange(npad, dtype=I32) % (NP - NN))
    src = jnp.concatenate([edge_index[0].astype(I32), loop, pad_s])
    dst = jnp.concatenate([edge_index[1].astype(I32), loop, pad_d])
    x = jnp.pad(x, ((0, NP - NN), (0, 0)))

    whl1 = jnp.concatenate([W1, Wl1], axis=1)
    wa1 = _wa_table(W1, a1s, a1d, 4, 256, 4)
    bs1 = (b1 + bl1)[None, :]

    whl2 = jnp.concatenate([W2, Wl2], axis=1)
    wa2 = _wa_table(W2, a2s, a2d, 4, 256, 4)
    bs2 = (b2 + bl2)[None, :]

    w3p = jnp.pad(W3.reshape(-1, 6, 121), ((0, 0), (0, 0), (0, 7))).reshape(-1, 768)
    wl3p = jnp.pad(Wl3, ((0, 0), (0, 7)))
    whl3 = jnp.concatenate(
        [w3p, wl3p, jnp.zeros((4 * 256, 2048 - 896), F32)], axis=1)
    wa3 = _wa_table(W3, a3s, a3d, 6, 121, 3)
    bs3 = jnp.pad(b3 + bl3, (0, 7))[None, :]

    meta12 = jnp.array([[0, 0, 1, 1, 2, 2, 3, 3] + [0] * 120], dtype=F32)
    meta3 = jnp.array([[0, 1, 2, 3, 4, 5, 0, 0] + [0] * 120], dtype=F32)

    h1 = _layer(x, whl1, wa1, bs1, src, dst, meta12, _fix12)
    h2 = _layer(h1, whl2, wa2, bs2, src, dst, meta12, _fix12)
    out = _layer(h2, whl3, wa3, bs3, src, dst, meta3, _fix3)
    return out


# speedup vs baseline: 1.0560x; 1.0560x over previous
import functools

import jax
import jax.numpy as jnp
from jax import lax
from jax.experimental import pallas as pl
from jax.experimental.pallas import tpu as pltpu
from jax.experimental.pallas import tpu_sc as plsc

NN = 10000
EE = 330000
EP = 331776
EB = 64
NC = 2
NS = 16
NP = 10240
RPT = NP // NS
F32 = jnp.float32
I32 = jnp.int32


def _mm_body(x_ref, w_ref, o_ref):
    o_ref[...] = jnp.dot(x_ref[...], w_ref[...], preferred_element_type=F32)


def _mm(x, w, bm=512):
    m, k = x.shape
    _, n = w.shape
    return pl.pallas_call(
        _mm_body,
        grid=(m // bm,),
        in_specs=[pl.BlockSpec((bm, k), lambda i: (i, 0)),
                  pl.BlockSpec((k, n), lambda i: (0, 0))],
        out_specs=pl.BlockSpec((bm, n), lambda i: (i, 0)),
        out_shape=jax.ShapeDtypeStruct((m, n), F32),
    )(x, w)


def _mma_body(x_ref, w_ref, a_ref, g_ref):
    r = jnp.dot(x_ref[...], w_ref[...], preferred_element_type=F32)
    a_ref[...] = r
    rmax = jnp.max(r, axis=0, keepdims=True)

    @pl.when(pl.program_id(0) == 0)
    def _():
        g_ref[...] = rmax

    @pl.when(pl.program_id(0) != 0)
    def _():
        g_ref[...] = jnp.maximum(g_ref[...], rmax)


def _mma(x, w, bm=512):
    m, k = x.shape
    return pl.pallas_call(
        _mma_body,
        grid=(m // bm,),
        in_specs=[pl.BlockSpec((bm, k), lambda i: (i, 0)),
                  pl.BlockSpec((k, 128), lambda i: (0, 0))],
        out_specs=[pl.BlockSpec((bm, 128), lambda i: (i, 0)),
                   pl.BlockSpec((1, 128), lambda i: (0, 0))],
        out_shape=[jax.ShapeDtypeStruct((m, 128), F32),
                   jax.ShapeDtypeStruct((1, 128), F32)],
    )(x, w)


def _phaseB_body(src_hbm, dst_hbm, asd_hbm, gmax_hbm,
                 ex_hbm,
                 srcb, dstb, srows, drows, exb, gb):
    cid = lax.axis_index("c")
    sid = lax.axis_index("s")
    wid = sid * NC + cid

    pltpu.sync_copy(gmax_hbm, gb)

    nb = EP // (NC * NS) // 32
    base0 = wid * (EP // (NC * NS))

    def batch(b, _):
        base = base0 + b * 32
        pltpu.sync_copy(src_hbm.at[pl.ds(base, 32)], srcb)
        pltpu.sync_copy(dst_hbm.at[pl.ds(base, 32)], dstb)
        pltpu.sync_copy(asd_hbm.at[srcb], srows)
        pltpu.sync_copy(asd_hbm.at[dstb], drows)
        gv = gb[0, pl.ds(0, 16)]

        def edge(e, _):
            asv = srows[e, pl.ds(0, 16)]
            adv = drows[e, pl.ds(16, 16)]
            z = asv + adv
            z = jnp.where(z >= 0.0, z, 0.2 * z)
            mm = gv + adv
            mm = jnp.where(mm >= 0.0, mm, 0.2 * mm)
            exb[e, :] = jnp.exp(z - mm)
            return _
        lax.fori_loop(0, 32, edge, None)

        pltpu.sync_copy(exb, ex_hbm.at[pl.ds(base, 32)])
        return _
    lax.fori_loop(0, nb, batch, None)


_phaseB = pl.kernel(
    _phaseB_body,
    out_type=jax.ShapeDtypeStruct((EP, 16), F32),
    mesh=plsc.VectorSubcoreMesh(core_axis_name="c", subcore_axis_name="s"),
    scratch_types=[
        pltpu.VMEM((32,), I32), pltpu.VMEM((32,), I32),
        pltpu.VMEM((32, 128), F32), pltpu.VMEM((32, 128), F32),
        pltpu.VMEM((32, 16), F32),
        pltpu.VMEM((1, 128), F32),
    ])


def _phaseC_body(src_hbm, dst_hbm, ex_hbm, xw_hbm, meta_hbm,
                 acc_hbm, den_hbm,
                 acc_sp, idxb, exrb, hrow0, hrow1, zb,
                 sg0, sg1, ss0, ss1):
    cid = lax.axis_index("c")
    sid = lax.axis_index("s")
    srcs = (idxb.at[0, 0], idxb.at[1, 0])
    dsts = (idxb.at[0, 1], idxb.at[1, 1])
    exrs = (exrb.at[0], exrb.at[1])
    hrows = (hrow0, hrow1)
    sgs = (sg0, sg1)
    sss = (ss0, ss1)

    def zz(i, _):
        for j in range(8):
            zb[i, pl.ds(j * 16, 16)] = jnp.zeros((16,), F32)
        return _
    lax.fori_loop(0, 4, zz, None)

    pltpu.sync_copy(meta_hbm, zb.at[pl.ds(4, 1)])
    metav = zb[4, pl.ds(0, 16)].astype(I32)

    nb = EP // NS // EB
    base0 = sid * (EP // NS)

    def zero_acc():
        for z5 in range(160):
            pltpu.sync_copy(zb.at[pl.ds(0, 4)],
                            acc_sp.at[pl.ds(sid * RPT + z5 * 4, 4)])

    def _bcast16(v, idx):
        if isinstance(idx, jax.Array) and idx.ndim == 1:
            ind = idx[:, None]
        else:
            ind = jnp.full((16, 1), idx, I32)
        return lax.gather(
            v, ind,
            lax.GatherDimensionNumbers(
                offset_dims=(), collapsed_slice_dims=(0,),
                start_index_map=(0,)),
            (1,), mode=lax.GatherScatterMode.PROMISE_IN_BOUNDS)

    for k in range(4):
        cs = cid * 4 + k
        lanevec = _bcast16(metav, cs)

        zero_acc()
        plsc.subcore_barrier()

        def loads(b, s):
            base = base0 + b * EB
            pltpu.sync_copy(src_hbm.at[pl.ds(base, EB)], srcs[s])
            pltpu.sync_copy(dst_hbm.at[pl.ds(base, EB)], dsts[s])
            pltpu.sync_copy(ex_hbm.at[pl.ds(base, EB)], exrs[s])
            for c in range(EB // 16):
                sl = pl.ds(c * 16, 16)
                srcs[s][sl] = srcs[s][sl] * 16 + cs

        def g_issue(s):
            pltpu.async_copy(xw_hbm.at[srcs[s]], hrows[s], sgs[s])

        def g_wait(s):
            pltpu.make_async_copy(xw_hbm.at[srcs[s]], hrows[s], sgs[s]).wait()

        def s_issue(s):
            pltpu.async_copy(hrows[s], acc_sp.at[dsts[s]], sss[s], add=True)

        def s_wait(s):
            pltpu.make_async_copy(
                hrows[s], acc_sp.at[dsts[s]], sss[s]).wait()

        def scale(s):
            exr = exrs[s]
            hr = hrows[s]

            def edge4(q, _):
                for u in range(4):
                    e = q * 4 + u
                    alpha = _bcast16(exr[e, :], lanevec)
                    for j in range(8):
                        sl = pl.ds(j * 16, 16)
                        hr[e, sl] = hr[e, sl] * alpha
                return _
            lax.fori_loop(0, EB // 4, edge4, None)

        loads(0, 0)
        g_issue(0)

        def pair(p, _):
            b = 2 * p

            @pl.when(p >= 1)
            def _():
                s_wait(1)
            loads(b + 1, 1)
            g_issue(1)
            g_wait(0)
            scale(0)
            s_issue(0)

            g_wait(1)
            scale(1)
            s_issue(1)
            s_wait(0)
            loads(b + 2, 0)
            g_issue(0)
            return _
        lax.fori_loop(0, (nb - 2) // 2, pair, None)

        s_wait(1)
        loads(nb - 1, 1)
        g_issue(1)
        g_wait(0)
        scale(0)
        s_issue(0)
        g_wait(1)
        scale(1)
        s_issue(1)
        s_wait(0)
        s_wait(1)

        plsc.subcore_barrier()
        pltpu.sync_copy(acc_sp.at[pl.ds(sid * RPT, RPT)],
                        acc_hbm.at[cs, pl.ds(sid * RPT, RPT)])
        plsc.subcore_barrier()

    def zh(i, _):
        for j in range(8):
            hrow0[i, pl.ds(j * 16, 16)] = jnp.zeros((16,), F32)
            hrow1[i, pl.ds(j * 16, 16)] = jnp.zeros((16,), F32)
        return _
    lax.fori_loop(0, EB, zh, None)
    zero_acc()
    plsc.subcore_barrier()

    nb2 = EP // (NC * NS) // EB
    base1 = cid * (EP // NC) + sid * (EP // (NC * NS))

    def dloads(b, s):
        base = base1 + b * EB
        pltpu.sync_copy(dst_hbm.at[pl.ds(base, EB)], dsts[s])
        pltpu.sync_copy(ex_hbm.at[pl.ds(base, EB)], exrs[s])

    def dfill(s):
        exr = exrs[s]
        hr = hrows[s]

        def de(e, _):
            hr[e, pl.ds(0, 16)] = exr[e, :]
            return _
        lax.fori_loop(0, EB, de, None)

    def s_issue2(s):
        pltpu.async_copy(hrows[s], acc_sp.at[dsts[s]], sss[s], add=True)

    def s_wait2(s):
        pltpu.make_async_copy(
            hrows[s], acc_sp.at[dsts[s]], sss[s]).wait()

    dloads(0, 0)
    dfill(0)
    s_issue2(0)

    def dpair(p, _):
        b = 2 * p

        @pl.when(p >= 1)
        def _():
            s_wait2(1)
        dloads(b + 1, 1)
        dfill(1)
        s_issue2(1)
        s_wait2(0)
        dloads(b + 2, 0)
        dfill(0)
        s_issue2(0)
        return _
    lax.fori_loop(0, (nb2 - 2) // 2, dpair, None)

    s_wait2(1)
    dloads(nb2 - 1, 1)
    dfill(1)
    s_issue2(1)
    s_wait2(0)
    s_wait2(1)

    plsc.subcore_barrier()
    pltpu.sync_copy(acc_sp.at[pl.ds(sid * RPT, RPT)],
                    den_hbm.at[cid, pl.ds(sid * RPT, RPT)])


_phaseC = pl.kernel(
    _phaseC_body,
    out_type=[jax.ShapeDtypeStruct((8, NP, 128), F32),
              jax.ShapeDtypeStruct((NC, NP, 128), F32)],
    mesh=plsc.VectorSubcoreMesh(core_axis_name="c", subcore_axis_name="s"),
    scratch_types=[
        pltpu.VMEM_SHARED((NP, 128), F32),
        pltpu.VMEM((2, 2, EB), I32),
        pltpu.VMEM((2, EB, 16), F32),
        pltpu.VMEM((EB, 128), F32), pltpu.VMEM((EB, 128), F32),
        pltpu.VMEM((5, 128), F32),
        pltpu.SemaphoreType.DMA, pltpu.SemaphoreType.DMA,
        pltpu.SemaphoreType.DMA, pltpu.SemaphoreType.DMA,
    ])


def _fix12_body(acc_ref, den_ref, lin_ref, b_ref, o_ref):
    den = den_ref[0] + den_ref[1] + 1e-16
    for s in range(8):
        h = s // 2
        d = den[:, h:h + 1]
        t = acc_ref[s] / d + lin_ref[:, s * 128:(s + 1) * 128] \
            + b_ref[:, s * 128:(s + 1) * 128]
        o_ref[:, s * 128:(s + 1) * 128] = jnp.where(t > 0.0, t, jnp.exp(t) - 1.0)


def _fix12(acc, den, xw, bsum, bm=512):
    return pl.pallas_call(
        _fix12_body,
        grid=(NP // bm,),
        in_specs=[pl.BlockSpec((8, bm, 128), lambda i: (0, i, 0)),
                  pl.BlockSpec((2, bm, 128), lambda i: (0, i, 0)),
                  pl.BlockSpec((bm, 1024), lambda i: (i, 1)),
                  pl.BlockSpec((1, 1024), lambda i: (0, 0))],
        out_specs=pl.BlockSpec((bm, 1024), lambda i: (i, 0)),
        out_shape=jax.ShapeDtypeStruct((NP, 1024), F32),
    )(acc, den, xw, bsum)


def _fix3_body(acc_ref, den_ref, lin_ref, b_ref, o_ref):
    den = den_ref[0] + den_ref[1] + 1e-16
    t = jnp.zeros(acc_ref.shape[1:], F32)
    for s in range(6):
        t = t + acc_ref[s] / den[:, s:s + 1]
    t = t * (1.0 / 6.0) + lin_ref[...] + b_ref[...]
    o_ref[...] = t[:, :121]


def _fix3(acc, den, xw, bsum, bm=400):
    return pl.pallas_call(
        _fix3_body,
        grid=(NN // bm,),
        in_specs=[pl.BlockSpec((6, bm, 128), lambda i: (0, i, 0)),
                  pl.BlockSpec((2, bm, 128), lambda i: (0, i, 0)),
                  pl.BlockSpec((bm, 128), lambda i: (i, 6)),
                  pl.BlockSpec((1, 128), lambda i: (0, 0))],
        out_specs=pl.BlockSpec((bm, 121), lambda i: (i, 0)),
        out_shape=jax.ShapeDtypeStruct((NN, 121), F32),
    )(acc, den, xw, bsum)


def _wa_table(W, a_s, a_d, heads, ch, reps):
    ws = (W.reshape(-1, heads, ch) * a_s[None]).sum(-1)
    wd = (W.reshape(-1, heads, ch) * a_d[None]).sum(-1)
    ws16 = jnp.tile(ws, (1, reps))[:, :16]
    wd16 = jnp.tile(wd, (1, reps))[:, :16]
    pad = jnp.zeros((W.shape[0], 96), F32)
    return jnp.concatenate([ws16, wd16, pad], axis=1)


def _layer(x, whl, wa, bsum, src, dst, meta, fix):
    xw = _mm(x, whl)
    asd, gmax = _mma(x, wa)
    ex = _phaseB(src, dst, asd, gmax)
    xw_v = xw.reshape(16 * NP, 128)
    acc, den = _phaseC(src, dst, ex, xw_v, meta)
    return fix(acc, den, xw, bsum)


def kernel(x, edge_index, W1, a1s, a1d, b1, Wl1, bl1, W2, a2s, a2d, b2,
           Wl2, bl2, W3, a3s, a3d, b3, Wl3, bl3):
    loop = jnp.arange(NN, dtype=I32)
    npad = EP - EE
    pad_s = (jnp.arange(npad, dtype=I32) * 97) % NN
    pad_d = NN + (jnp.arange(npad, dtype=I32) % (NP - NN))
    src = jnp.concatenate([edge_index[0].astype(I32), loop, pad_s])
    dst = jnp.concatenate([edge_index[1].astype(I32), loop, pad_d])
    x = jnp.pad(x, ((0, NP - NN), (0, 0)))

    whl1 = jnp.concatenate([W1, Wl1], axis=1)
    wa1 = _wa_table(W1, a1s, a1d, 4, 256, 4)
    bs1 = (b1 + bl1)[None, :]

    whl2 = jnp.concatenate([W2, Wl2], axis=1)
    wa2 = _wa_table(W2, a2s, a2d, 4, 256, 4)
    bs2 = (b2 + bl2)[None, :]

    w3p = jnp.pad(W3.reshape(-1, 6, 121), ((0, 0), (0, 0), (0, 7))).reshape(-1, 768)
    wl3p = jnp.pad(Wl3, ((0, 0), (0, 7)))
    whl3 = jnp.concatenate(
        [w3p, wl3p, jnp.zeros((4 * 256, 2048 - 896), F32)], axis=1)
    wa3 = _wa_table(W3, a3s, a3d, 6, 121, 3)
    bs3 = jnp.pad(b3 + bl3, (0, 7))[None, :]

    meta12 = jnp.array([[0, 0, 1, 1, 2, 2, 3, 3] + [0] * 120], dtype=F32)
    meta3 = jnp.array([[0, 1, 2, 3, 4, 5, 0, 0] + [0] * 120], dtype=F32)

    h1 = _layer(x, whl1, wa1, bs1, src, dst, meta12, _fix12)
    h2 = _layer(h1, whl2, wa2, bs2, src, dst, meta12, _fix12)
    out = _layer(h2, whl3, wa3, bs3, src, dst, meta3, _fix3)
    return out


# speedup vs baseline: 1.0920x; 1.0341x over previous
import functools

import jax
import jax.numpy as jnp
from jax import lax
from jax.experimental import pallas as pl
from jax.experimental.pallas import tpu as pltpu
from jax.experimental.pallas import tpu_sc as plsc

NN = 10000
EE = 330000
EP = 331776
EB = 128
NC = 2
NS = 16
NP = 10240
RPT = NP // NS
F32 = jnp.float32
I32 = jnp.int32


def _mm_body(x_ref, w_ref, o_ref):
    o_ref[...] = jnp.dot(x_ref[...], w_ref[...], preferred_element_type=F32)


def _mm(x, w, bm=512):
    m, k = x.shape
    _, n = w.shape
    return pl.pallas_call(
        _mm_body,
        grid=(m // bm,),
        in_specs=[pl.BlockSpec((bm, k), lambda i: (i, 0)),
                  pl.BlockSpec((k, n), lambda i: (0, 0))],
        out_specs=pl.BlockSpec((bm, n), lambda i: (i, 0)),
        out_shape=jax.ShapeDtypeStruct((m, n), F32),
    )(x, w)


def _mma_body(x_ref, w_ref, a_ref, g_ref):
    r = jnp.dot(x_ref[...], w_ref[...], preferred_element_type=F32)
    a_ref[...] = r
    rmax = jnp.max(r, axis=0, keepdims=True)

    @pl.when(pl.program_id(0) == 0)
    def _():
        g_ref[...] = rmax

    @pl.when(pl.program_id(0) != 0)
    def _():
        g_ref[...] = jnp.maximum(g_ref[...], rmax)


def _mma(x, w, bm=512):
    m, k = x.shape
    return pl.pallas_call(
        _mma_body,
        grid=(m // bm,),
        in_specs=[pl.BlockSpec((bm, k), lambda i: (i, 0)),
                  pl.BlockSpec((k, 128), lambda i: (0, 0))],
        out_specs=[pl.BlockSpec((bm, 128), lambda i: (i, 0)),
                   pl.BlockSpec((1, 128), lambda i: (0, 0))],
        out_shape=[jax.ShapeDtypeStruct((m, 128), F32),
                   jax.ShapeDtypeStruct((1, 128), F32)],
    )(x, w)


def _phaseB_body(src_hbm, dst_hbm, asd_hbm, gmax_hbm,
                 ex_hbm,
                 srcb, dstb, srows, drows, exb, gb):
    cid = lax.axis_index("c")
    sid = lax.axis_index("s")
    wid = sid * NC + cid

    pltpu.sync_copy(gmax_hbm, gb)

    nb = EP // (NC * NS) // 32
    base0 = wid * (EP // (NC * NS))

    def batch(b, _):
        base = base0 + b * 32
        pltpu.sync_copy(src_hbm.at[pl.ds(base, 32)], srcb)
        pltpu.sync_copy(dst_hbm.at[pl.ds(base, 32)], dstb)
        pltpu.sync_copy(asd_hbm.at[srcb], srows)
        pltpu.sync_copy(asd_hbm.at[dstb], drows)
        gv = gb[0, pl.ds(0, 16)]

        def edge(e, _):
            asv = srows[e, pl.ds(0, 16)]
            adv = drows[e, pl.ds(16, 16)]
            z = asv + adv
            z = jnp.where(z >= 0.0, z, 0.2 * z)
            mm = gv + adv
            mm = jnp.where(mm >= 0.0, mm, 0.2 * mm)
            exb[e, :] = jnp.exp(z - mm)
            return _
        lax.fori_loop(0, 32, edge, None)

        pltpu.sync_copy(exb, ex_hbm.at[pl.ds(base, 32)])
        return _
    lax.fori_loop(0, nb, batch, None)


_phaseB = pl.kernel(
    _phaseB_body,
    out_type=jax.ShapeDtypeStruct((EP, 16), F32),
    mesh=plsc.VectorSubcoreMesh(core_axis_name="c", subcore_axis_name="s"),
    scratch_types=[
        pltpu.VMEM((32,), I32), pltpu.VMEM((32,), I32),
        pltpu.VMEM((32, 128), F32), pltpu.VMEM((32, 128), F32),
        pltpu.VMEM((32, 16), F32),
        pltpu.VMEM((1, 128), F32),
    ])


def _phaseC_body(src_hbm, dst_hbm, ex_hbm, xw_hbm, meta_hbm,
                 acc_hbm, den_hbm,
                 acc_sp, srcb, dstb, gix, exrows, hrows, zb, metab):
    cid = lax.axis_index("c")
    sid = lax.axis_index("s")

    def zz(i, _):
        for j in range(8):
            zb[i, pl.ds(j * 16, 16)] = jnp.zeros((16,), F32)
        return _
    lax.fori_loop(0, 32, zz, None)

    pltpu.sync_copy(meta_hbm, metab)
    metav = metab[0, pl.ds(0, 16)].astype(I32)

    nb = EP // NS // EB
    base0 = sid * (EP // NS)

    def zero_acc():
        for z5 in range(20):
            pltpu.sync_copy(zb, acc_sp.at[pl.ds(sid * RPT + z5 * 32, 32)])

    def _bcast16(v, idx):
        if isinstance(idx, jax.Array) and idx.ndim == 1:
            ind = idx[:, None]
        else:
            ind = jnp.full((16, 1), idx, I32)
        return lax.gather(
            v, ind,
            lax.GatherDimensionNumbers(
                offset_dims=(), collapsed_slice_dims=(0,),
                start_index_map=(0,)),
            (1,), mode=lax.GatherScatterMode.PROMISE_IN_BOUNDS)

    for k in range(4):
        cs = cid * 4 + k
        lanevec = _bcast16(metav, cs)

        zero_acc()
        plsc.subcore_barrier()

        def batch(b, _):
            base = base0 + b * EB
            pltpu.sync_copy(src_hbm.at[pl.ds(base, EB)], srcb)
            pltpu.sync_copy(dst_hbm.at[pl.ds(base, EB)], dstb)
            pltpu.sync_copy(ex_hbm.at[pl.ds(base, EB)], exrows)
            for c in range(EB // 16):
                gix[pl.ds(c * 16, 16)] = srcb[pl.ds(c * 16, 16)] * 16 + cs
            pltpu.sync_copy(xw_hbm.at[gix], hrows)

            def edge4(q, _):
                for u in range(4):
                    e = q * 4 + u
                    alpha = _bcast16(exrows[e, :], lanevec)
                    for j in range(8):
                        sl = pl.ds(j * 16, 16)
                        hrows[e, sl] = hrows[e, sl] * alpha
                return _
            lax.fori_loop(0, EB // 4, edge4, None)

            pltpu.sync_copy(hrows, acc_sp.at[dstb], add=True)
            return _
        lax.fori_loop(0, nb, batch, None)

        plsc.subcore_barrier()
        pltpu.sync_copy(acc_sp.at[pl.ds(sid * RPT, RPT)],
                        acc_hbm.at[cs, pl.ds(sid * RPT, RPT)])
        plsc.subcore_barrier()

    def zh(i, _):
        for j in range(8):
            hrows[i, pl.ds(j * 16, 16)] = jnp.zeros((16,), F32)
        return _
    lax.fori_loop(0, EB, zh, None)
    zero_acc()
    plsc.subcore_barrier()

    nb2 = EP // (NC * NS) // EB
    base1 = cid * (EP // NC) + sid * (EP // (NC * NS))

    def dbatch(b, _):
        base = base1 + b * EB
        pltpu.sync_copy(dst_hbm.at[pl.ds(base, EB)], dstb)
        pltpu.sync_copy(ex_hbm.at[pl.ds(base, EB)], exrows)

        def de(e, _):
            hrows[e, pl.ds(0, 16)] = exrows[e, :]
            return _
        lax.fori_loop(0, EB, de, None)

        pltpu.sync_copy(hrows, acc_sp.at[dstb], add=True)
        return _
    lax.fori_loop(0, nb2, dbatch, None)

    plsc.subcore_barrier()
    pltpu.sync_copy(acc_sp.at[pl.ds(sid * RPT, RPT)],
                    den_hbm.at[cid, pl.ds(sid * RPT, RPT)])


_phaseC = pl.kernel(
    _phaseC_body,
    out_type=[jax.ShapeDtypeStruct((8, NP, 128), F32),
              jax.ShapeDtypeStruct((NC, NP, 128), F32)],
    mesh=plsc.VectorSubcoreMesh(core_axis_name="c", subcore_axis_name="s"),
    scratch_types=[
        pltpu.VMEM_SHARED((NP, 128), F32),
        pltpu.VMEM((EB,), I32), pltpu.VMEM((EB,), I32),
        pltpu.VMEM((EB,), I32),
        pltpu.VMEM((EB, 16), F32),
        pltpu.VMEM((EB, 128), F32),
        pltpu.VMEM((32, 128), F32),
        pltpu.VMEM((1, 128), F32),
    ])


def _fix12_body(acc_ref, den_ref, lin_ref, b_ref, o_ref):
    den = den_ref[0] + den_ref[1] + 1e-16
    for s in range(8):
        h = s // 2
        d = den[:, h:h + 1]
        t = acc_ref[s] / d + lin_ref[:, s * 128:(s + 1) * 128] \
            + b_ref[:, s * 128:(s + 1) * 128]
        o_ref[:, s * 128:(s + 1) * 128] = jnp.where(t > 0.0, t, jnp.exp(t) - 1.0)


def _fix12(acc, den, xw, bsum, bm=512):
    return pl.pallas_call(
        _fix12_body,
        grid=(NP // bm,),
        in_specs=[pl.BlockSpec((8, bm, 128), lambda i: (0, i, 0)),
                  pl.BlockSpec((2, bm, 128), lambda i: (0, i, 0)),
                  pl.BlockSpec((bm, 1024), lambda i: (i, 1)),
                  pl.BlockSpec((1, 1024), lambda i: (0, 0))],
        out_specs=pl.BlockSpec((bm, 1024), lambda i: (i, 0)),
        out_shape=jax.ShapeDtypeStruct((NP, 1024), F32),
    )(acc, den, xw, bsum)


def _fix3_body(acc_ref, den_ref, lin_ref, b_ref, o_ref):
    den = den_ref[0] + den_ref[1] + 1e-16
    t = jnp.zeros(acc_ref.shape[1:], F32)
    for s in range(6):
        t = t + acc_ref[s] / den[:, s:s + 1]
    t = t * (1.0 / 6.0) + lin_ref[...] + b_ref[...]
    o_ref[...] = t[:, :121]


def _fix3(acc, den, xw, bsum, bm=400):
    return pl.pallas_call(
        _fix3_body,
        grid=(NN // bm,),
        in_specs=[pl.BlockSpec((6, bm, 128), lambda i: (0, i, 0)),
                  pl.BlockSpec((2, bm, 128), lambda i: (0, i, 0)),
                  pl.BlockSpec((bm, 128), lambda i: (i, 6)),
                  pl.BlockSpec((1, 128), lambda i: (0, 0))],
        out_specs=pl.BlockSpec((bm, 121), lambda i: (i, 0)),
        out_shape=jax.ShapeDtypeStruct((NN, 121), F32),
    )(acc, den, xw, bsum)


def _wa_table(W, a_s, a_d, heads, ch, reps):
    ws = (W.reshape(-1, heads, ch) * a_s[None]).sum(-1)
    wd = (W.reshape(-1, heads, ch) * a_d[None]).sum(-1)
    ws16 = jnp.tile(ws, (1, reps))[:, :16]
    wd16 = jnp.tile(wd, (1, reps))[:, :16]
    pad = jnp.zeros((W.shape[0], 96), F32)
    return jnp.concatenate([ws16, wd16, pad], axis=1)


def _layer(x, whl, wa, bsum, src, dst, meta, fix):
    xw = _mm(x, whl)
    asd, gmax = _mma(x, wa)
    ex = _phaseB(src, dst, asd, gmax)
    xw_v = xw.reshape(16 * NP, 128)
    acc, den = _phaseC(src, dst, ex, xw_v, meta)
    return fix(acc, den, xw, bsum)


def kernel(x, edge_index, W1, a1s, a1d, b1, Wl1, bl1, W2, a2s, a2d, b2,
           Wl2, bl2, W3, a3s, a3d, b3, Wl3, bl3):
    loop = jnp.arange(NN, dtype=I32)
    npad = EP - EE
    pad_s = (jnp.arange(npad, dtype=I32) * 97) % NN
    pad_d = NN + (jnp.arange(npad, dtype=I32) % (NP - NN))
    src = jnp.concatenate([edge_index[0].astype(I32), loop, pad_s])
    dst = jnp.concatenate([edge_index[1].astype(I32), loop, pad_d])
    x = jnp.pad(x, ((0, NP - NN), (0, 0)))

    whl1 = jnp.concatenate([W1, Wl1], axis=1)
    wa1 = _wa_table(W1, a1s, a1d, 4, 256, 4)
    bs1 = (b1 + bl1)[None, :]

    whl2 = jnp.concatenate([W2, Wl2], axis=1)
    wa2 = _wa_table(W2, a2s, a2d, 4, 256, 4)
    bs2 = (b2 + bl2)[None, :]

    w3p = jnp.pad(W3.reshape(-1, 6, 121), ((0, 0), (0, 0), (0, 7))).reshape(-1, 768)
    wl3p = jnp.pad(Wl3, ((0, 0), (0, 7)))
    whl3 = jnp.concatenate(
        [w3p, wl3p, jnp.zeros((4 * 256, 2048 - 896), F32)], axis=1)
    wa3 = _wa_table(W3, a3s, a3d, 6, 121, 3)
    bs3 = jnp.pad(b3 + bl3, (0, 7))[None, :]

    meta12 = jnp.array([[0, 0, 1, 1, 2, 2, 3, 3] + [0] * 120], dtype=F32)
    meta3 = jnp.array([[0, 1, 2, 3, 4, 5, 0, 0] + [0] * 120], dtype=F32)

    h1 = _layer(x, whl1, wa1, bs1, src, dst, meta12, _fix12)
    h2 = _layer(h1, whl2, wa2, bs2, src, dst, meta12, _fix12)
    out = _layer(h2, whl3, wa3, bs3, src, dst, meta3, _fix3)
    return out
